# baseline (speedup 1.0000x reference)
import functools

import jax
import jax.numpy as jnp
from jax import lax
from jax.experimental import pallas as pl
from jax.experimental.pallas import tpu as pltpu
from jax.experimental.pallas import tpu_sc as plsc

EMB_D = 128
NG = 64
NN = 512
NE = 8192
LANES = 16
NC, NS = 2, 16
NW = NC * NS

NHALF = 2
HGN = NG // NHALF
H_ROWS_W = HGN * NN // NW
H_GCH = H_ROWS_W // 128

TPG = 4
RPT = NN // TPG
GPP = NW // TPG
H_NPASS = HGN // GPP

GPS = 2

_MESH = plsc.VectorSubcoreMesh(
    core_axis_name="c", subcore_axis_name="s", num_cores=NC, num_subcores=NS)
_SC_PARAMS = pltpu.CompilerParams(needs_layout_passes=False)


def _make_gather(half):
    @functools.partial(
        pl.kernel,
        out_type=jax.ShapeDtypeStruct((HGN * NN, EMB_D), jnp.float32),
        mesh=_MESH,
        scratch_types=[
            pltpu.VMEM((H_GCH, 128), jnp.int32),
            pltpu.VMEM((2, 128, EMB_D), jnp.float32),
            pltpu.SemaphoreType.DMA,
            pltpu.SemaphoreType.DMA,
        ],
        compiler_params=_SC_PARAMS,
    )
    def _gather(nt_hbm, table_hbm, out_hbm, idx_v, rows_v, sem0, sem1):
        wid = lax.axis_index("s") * NC + lax.axis_index("c")
        base = wid * H_ROWS_W
        nt_row0 = half * (HGN * NN // 128) + wid * H_GCH
        pltpu.sync_copy(nt_hbm.at[pl.ds(nt_row0, H_GCH)], idx_v)
        sems = (sem0, sem1)
        cps = [None, None]
        cps[0] = pltpu.async_copy(
            table_hbm.at[idx_v.at[0]], rows_v.at[0], sems[0])
        for j in range(H_GCH):
            nj = j + 1
            if nj < H_GCH:
                cps[nj % 2] = pltpu.async_copy(
                    table_hbm.at[idx_v.at[nj]], rows_v.at[nj % 2], sems[nj % 2])
            cps[j % 2].wait()
            pltpu.sync_copy(
                rows_v.at[j % 2], out_hbm.at[pl.ds(base + j * 128, 128)])

    return _gather


def _make_adjacency(half):
    @functools.partial(
        pl.kernel,
        out_type=jax.ShapeDtypeStruct((HGN * NN * NN // 2,), jnp.int32),
        mesh=_MESH,
        scratch_types=[
            pltpu.VMEM((NE,), jnp.int32),
            pltpu.VMEM((NE,), jnp.int32),
            pltpu.VMEM((RPT * NN // 2,), jnp.int32),
        ],
        compiler_params=_SC_PARAMS,
    )
    def _adjacency(edges_hbm, out_hbm, src_v, dst_v, cblk_v):
        wid = lax.axis_index("s") * NC + lax.axis_index("c")
        g_off = wid // TPG
        lo = (wid % TPG) * RPT
        zeros = jnp.zeros((LANES,), jnp.int32)
        iota = lax.iota(jnp.int32, LANES)
        for p in range(H_NPASS):
            gl = p * GPP + g_off
            pltpu.sync_copy(edges_hbm.at[half * HGN + gl, 0], src_v)
            pltpu.sync_copy(edges_hbm.at[half * HGN + gl, 1], dst_v)

            @plsc.parallel_loop(0, RPT * NN // (2 * LANES), unroll=8)
            def _zero(i):
                cblk_v[pl.ds(i * LANES, LANES)] = zeros

            for rr in range(RPT // LANES):
                rloc = iota + rr * LANES
                widx = (rloc >> 1) * NN + rloc + lo
                val = jnp.where((rloc & 1) == 1, 1 << 16, 1)
                plsc.addupdate_scatter(cblk_v, [widx], val)

            @plsc.parallel_loop(0, NE // LANES, unroll=8)
            def _edges(i):
                s = src_v[pl.ds(i * LANES, LANES)]
                d = dst_v[pl.ds(i * LANES, LANES)]
                m = (d >= lo) & (d < lo + RPT)
                row = jnp.where(m, d - lo, 0)
                flat = row * NN + s
                widx = (row >> 1) * NN + s
                odd = (row & 1) == 1
                m_e = m & jnp.logical_not(odd)
                m_o = m & odd
                cnt_e, last_e = plsc.scan_count(flat, m_e)
                plsc.addupdate_scatter(cblk_v, [widx], cnt_e, mask=last_e)
                cnt_o, last_o = plsc.scan_count(flat, m_o)
                plsc.addupdate_scatter(
                    cblk_v, [widx], cnt_o << 16, mask=last_o)

            lo2 = (wid % TPG) * (RPT // 2)
            pltpu.sync_copy(
                cblk_v,
                out_hbm.at[pl.ds(gl * (NN * NN // 2) + lo2 * NN,
                                 RPT * NN // 2)])

    return _adjacency


_GATHER_K = tuple(_make_gather(h) for h in range(NHALF))
_ADJ_K = tuple(_make_adjacency(h) for h in range(NHALF))


def _tc_half_body(xr, cg, w1, b1, w2, b2, w1blk, out_ref):
    g = pl.program_id(0)
    hp = jax.lax.Precision.DEFAULT

    def dot_t(a, b):
        return lax.dot_general(a, b, (((1,), (1,)), ((), ())), precision=hp,
                               preferred_element_type=jnp.float32)

    def dot_n(a, b):
        return lax.dot_general(a, b, (((1,), (0,)), ((), ())), precision=hp,
                               preferred_element_type=jnp.float32)

    def graph_feat(cw, xraw):
        clo = (cw & 0xFFFF).astype(jnp.float32)
        chi = lax.shift_right_logical(cw, 16).astype(jnp.float32)
        wsum = jnp.sum(cw, axis=1, keepdims=True)
        deg_e = (wsum & 0xFFFF).astype(jnp.float32)
        deg_o = lax.shift_right_logical(wsum, 16).astype(jnp.float32)
        dis_e = lax.rsqrt(deg_e)
        dis_o = lax.rsqrt(deg_o)
        dis = jnp.stack([dis_e, dis_o], axis=1).reshape(NN, 1)
        x = jnp.tanh(xraw)
        h1 = dot_t(x, w1[...])
        v1 = dis * h1
        x2_e = jnp.tanh(dis_e * dot_n(clo, v1) + b1[...])
        x2_o = jnp.tanh(dis_o * dot_n(chi, v1) + b1[...])
        v2 = jnp.stack(
            [dis_e * dot_t(x2_e, w2[...]), dis_o * dot_t(x2_o, w2[...])],
            axis=1).reshape(NN, EMB_D)
        x3_e = jnp.tanh(dis_e * dot_n(clo, v2) + b2[...])
        x3_o = jnp.tanh(dis_o * dot_n(chi, v2) + b2[...])
        return (jnp.sum(x3_e, axis=0, keepdims=True)
                + jnp.sum(x3_o, axis=0, keepdims=True)) * (1.0 / NN)

    w1b = w1blk[...]
    contrib = jnp.zeros((1, EMB_D), jnp.float32)
    for k in range(GPS):
        feat = graph_feat(cg[k], xr[pl.ds(k * NN, NN), :])
        contrib += dot_t(feat, w1b[:, k * EMB_D:(k + 1) * EMB_D])

    @pl.when(g == 0)
    def _():
        out_ref[...] = jnp.zeros((1, EMB_D), jnp.float32)

    out_ref[...] += contrib


def _tc_half(half, xraw, cmat, conv1_W, b1, conv2_W, b2, W1_w):
    w1_blk_off = half * (HGN // GPS)
    return pl.pallas_call(
        _tc_half_body,
        grid=(HGN // GPS,),
        in_specs=[
            pl.BlockSpec((GPS * NN, EMB_D), lambda g: (g, 0)),
            pl.BlockSpec((GPS, NN // 2, NN), lambda g: (g, 0, 0)),
            pl.BlockSpec((EMB_D, EMB_D), lambda g: (0, 0)),
            pl.BlockSpec((1, EMB_D), lambda g: (0, 0)),
            pl.BlockSpec((EMB_D, EMB_D), lambda g: (0, 0)),
            pl.BlockSpec((1, EMB_D), lambda g: (0, 0)),
            pl.BlockSpec((EMB_D, GPS * EMB_D),
                         lambda g: (0, g + w1_blk_off)),
        ],
        out_specs=pl.BlockSpec((1, EMB_D), lambda g: (0, 0)),
        out_shape=jax.ShapeDtypeStruct((1, EMB_D), jnp.float32),
        compiler_params=pltpu.CompilerParams(
            dimension_semantics=("arbitrary",)),
    )(xraw, cmat, conv1_W, b1, conv2_W, b2, W1_w)


def _head_body(accs, w1bias, w2w, w2bias, out_ref):
    hh = jnp.tanh(jnp.sum(accs[...], axis=0, keepdims=True) + w1bias[...])
    oo = jnp.sum(hh * w2w[...], axis=1, keepdims=True) + w2bias[...]
    out_ref[...] = jnp.tanh(oo)


def _head(accs, W1b, W2_w, W2b):
    return pl.pallas_call(
        _head_body,
        out_shape=jax.ShapeDtypeStruct((1, 1), jnp.float32),
    )(accs, W1b, W2_w, W2b)


def kernel(node_types, edge_indices, id_embed, conv1_W, conv1_b, conv2_W,
           conv2_b, W1_w, W1_b, W2_w, W2_b):
    nt2 = node_types.astype(jnp.int32).reshape(NG * NN // 128, 128)
    edges = edge_indices.astype(jnp.int32)
    b1 = conv1_b.reshape(1, EMB_D)
    b2 = conv2_b.reshape(1, EMB_D)
    accs = []
    for half in range(NHALF):
        xraw = _GATHER_K[half](nt2, id_embed)
        cmat = _ADJ_K[half](edges).reshape(HGN, NN // 2, NN)
        accs.append(
            _tc_half(half, xraw, cmat, conv1_W, b1, conv2_W, b2, W1_w))
    out = _head(jnp.concatenate(accs, axis=0), W1_b.reshape(1, EMB_D), W2_w,
                W2_b.reshape(1, 1))
    return jnp.squeeze(out)


# speedup vs baseline: 1.0447x; 1.0447x over previous
import functools

import jax
import jax.numpy as jnp
from jax import lax
from jax.experimental import pallas as pl
from jax.experimental.pallas import tpu as pltpu
from jax.experimental.pallas import tpu_sc as plsc

EMB_D = 128
NG = 64
NN = 512
NE = 8192
LANES = 16
NC, NS = 2, 16
NW = NC * NS

NHALF = 2
HGN = NG // NHALF
H_ROWS_W = HGN * NN // NW
H_GCH = H_ROWS_W // 128

TPG = 4
RPT = NN // TPG
GPP = NW // TPG
H_NPASS = HGN // GPP

GPS = 2

_MESH = plsc.VectorSubcoreMesh(
    core_axis_name="c", subcore_axis_name="s", num_cores=NC, num_subcores=NS)
_SC_PARAMS = pltpu.CompilerParams(needs_layout_passes=False)


def _make_gather(half):
    @functools.partial(
        pl.kernel,
        out_type=jax.ShapeDtypeStruct((HGN * NN, EMB_D), jnp.float32),
        mesh=_MESH,
        scratch_types=[
            pltpu.VMEM((H_GCH, 128), jnp.int32),
            pltpu.VMEM((2, 128, EMB_D), jnp.float32),
            pltpu.SemaphoreType.DMA,
            pltpu.SemaphoreType.DMA,
        ],
        compiler_params=_SC_PARAMS,
    )
    def _gather(nt_hbm, table_hbm, out_hbm, idx_v, rows_v, sem0, sem1):
        wid = lax.axis_index("s") * NC + lax.axis_index("c")
        base = wid * H_ROWS_W
        nt_row0 = half * (HGN * NN // 128) + wid * H_GCH
        pltpu.sync_copy(nt_hbm.at[pl.ds(nt_row0, H_GCH)], idx_v)
        sems = (sem0, sem1)
        cps = [None, None]
        cps[0] = pltpu.async_copy(
            table_hbm.at[idx_v.at[0]], rows_v.at[0], sems[0])
        for j in range(H_GCH):
            nj = j + 1
            if nj < H_GCH:
                cps[nj % 2] = pltpu.async_copy(
                    table_hbm.at[idx_v.at[nj]], rows_v.at[nj % 2], sems[nj % 2])
            cps[j % 2].wait()
            pltpu.sync_copy(
                rows_v.at[j % 2], out_hbm.at[pl.ds(base + j * 128, 128)])

    return _gather


def _make_adjacency(half):
    @functools.partial(
        pl.kernel,
        out_type=jax.ShapeDtypeStruct((HGN * NN * NN // 2,), jnp.int32),
        mesh=_MESH,
        scratch_types=[
            pltpu.VMEM((NE,), jnp.int32),
            pltpu.VMEM((NE,), jnp.int32),
            pltpu.VMEM((RPT * NN // 2,), jnp.int32),
        ],
        compiler_params=_SC_PARAMS,
    )
    def _adjacency(edges_hbm, out_hbm, src_v, dst_v, cblk_v):
        wid = lax.axis_index("s") * NC + lax.axis_index("c")
        g_off = wid // TPG
        lo = (wid % TPG) * RPT
        zeros = jnp.zeros((LANES,), jnp.int32)
        iota = lax.iota(jnp.int32, LANES)
        for p in range(H_NPASS):
            gl = p * GPP + g_off
            pltpu.sync_copy(edges_hbm.at[half * HGN + gl, 0], src_v)
            pltpu.sync_copy(edges_hbm.at[half * HGN + gl, 1], dst_v)

            @plsc.parallel_loop(0, RPT * NN // (2 * LANES), unroll=8)
            def _zero(i):
                cblk_v[pl.ds(i * LANES, LANES)] = zeros

            for rr in range(RPT // LANES):
                rloc = iota + rr * LANES
                widx = (rloc >> 1) * NN + rloc + lo
                val = jnp.where((rloc & 1) == 1, 1 << 16, 1)
                plsc.addupdate_scatter(cblk_v, [widx], val)

            @plsc.parallel_loop(0, NE // LANES, unroll=4)
            def _edges(i):
                s = src_v[pl.ds(i * LANES, LANES)]
                d = dst_v[pl.ds(i * LANES, LANES)]
                m = (d >= lo) & (d < lo + RPT)
                row = jnp.where(m, d - lo, 0)
                flat = row * NN + s
                widx = (row >> 1) * NN + s
                odd = (row & 1) == 1
                m_e = m & jnp.logical_not(odd)
                m_o = m & odd
                cnt_e, last_e = plsc.scan_count(flat, m_e)
                plsc.addupdate_scatter(cblk_v, [widx], cnt_e, mask=last_e)
                cnt_o, last_o = plsc.scan_count(flat, m_o)
                plsc.addupdate_scatter(
                    cblk_v, [widx], cnt_o << 16, mask=last_o)

            lo2 = (wid % TPG) * (RPT // 2)
            pltpu.sync_copy(
                cblk_v,
                out_hbm.at[pl.ds(gl * (NN * NN // 2) + lo2 * NN,
                                 RPT * NN // 2)])

    return _adjacency


_GATHER_K = tuple(_make_gather(h) for h in range(NHALF))
_ADJ_K = tuple(_make_adjacency(h) for h in range(NHALF))


def _tc_half_body(xr, cg, w1, b1, w2, b2, w1blk, out_ref):
    g = pl.program_id(0)
    hp = jax.lax.Precision.DEFAULT

    def dot_t(a, b):
        return lax.dot_general(a, b, (((1,), (1,)), ((), ())), precision=hp,
                               preferred_element_type=jnp.float32)

    def dot_n(a, b):
        return lax.dot_general(a, b, (((1,), (0,)), ((), ())), precision=hp,
                               preferred_element_type=jnp.float32)

    def graph_feat(cw, xraw):
        clo = (cw & 0xFFFF).astype(jnp.float32)
        chi = lax.shift_right_logical(cw, 16).astype(jnp.float32)
        wsum = jnp.sum(cw, axis=1, keepdims=True)
        deg_e = (wsum & 0xFFFF).astype(jnp.float32)
        deg_o = lax.shift_right_logical(wsum, 16).astype(jnp.float32)
        dis_e = lax.rsqrt(deg_e)
        dis_o = lax.rsqrt(deg_o)
        dis = jnp.stack([dis_e, dis_o], axis=1).reshape(NN, 1)
        x = jnp.tanh(xraw)
        h1 = dot_t(x, w1[...])
        v1 = dis * h1
        x2_e = jnp.tanh(dis_e * dot_n(clo, v1) + b1[...])
        x2_o = jnp.tanh(dis_o * dot_n(chi, v1) + b1[...])
        v2 = jnp.stack(
            [dis_e * dot_t(x2_e, w2[...]), dis_o * dot_t(x2_o, w2[...])],
            axis=1).reshape(NN, EMB_D)
        x3_e = jnp.tanh(dis_e * dot_n(clo, v2) + b2[...])
        x3_o = jnp.tanh(dis_o * dot_n(chi, v2) + b2[...])
        return (jnp.sum(x3_e, axis=0, keepdims=True)
                + jnp.sum(x3_o, axis=0, keepdims=True)) * (1.0 / NN)

    w1b = w1blk[...]
    contrib = jnp.zeros((1, EMB_D), jnp.float32)
    for k in range(GPS):
        feat = graph_feat(cg[k], xr[pl.ds(k * NN, NN), :])
        contrib += dot_t(feat, w1b[:, k * EMB_D:(k + 1) * EMB_D])

    @pl.when(g == 0)
    def _():
        out_ref[...] = jnp.zeros((1, EMB_D), jnp.float32)

    out_ref[...] += contrib


def _tc_half(half, xraw, cmat, conv1_W, b1, conv2_W, b2, W1_w):
    w1_blk_off = half * (HGN // GPS)
    return pl.pallas_call(
        _tc_half_body,
        grid=(HGN // GPS,),
        in_specs=[
            pl.BlockSpec((GPS * NN, EMB_D), lambda g: (g, 0)),
            pl.BlockSpec((GPS, NN // 2, NN), lambda g: (g, 0, 0)),
            pl.BlockSpec((EMB_D, EMB_D), lambda g: (0, 0)),
            pl.BlockSpec((1, EMB_D), lambda g: (0, 0)),
            pl.BlockSpec((EMB_D, EMB_D), lambda g: (0, 0)),
            pl.BlockSpec((1, EMB_D), lambda g: (0, 0)),
            pl.BlockSpec((EMB_D, GPS * EMB_D),
                         lambda g: (0, g + w1_blk_off)),
        ],
        out_specs=pl.BlockSpec((1, EMB_D), lambda g: (0, 0)),
        out_shape=jax.ShapeDtypeStruct((1, EMB_D), jnp.float32),
        compiler_params=pltpu.CompilerParams(
            dimension_semantics=("arbitrary",)),
    )(xraw, cmat, conv1_W, b1, conv2_W, b2, W1_w)


def _head_body(accs, w1bias, w2w, w2bias, out_ref):
    hh = jnp.tanh(jnp.sum(accs[...], axis=0, keepdims=True) + w1bias[...])
    oo = jnp.sum(hh * w2w[...], axis=1, keepdims=True) + w2bias[...]
    out_ref[...] = jnp.tanh(oo)


def _head(accs, W1b, W2_w, W2b):
    return pl.pallas_call(
        _head_body,
        out_shape=jax.ShapeDtypeStruct((1, 1), jnp.float32),
    )(accs, W1b, W2_w, W2b)


def kernel(node_types, edge_indices, id_embed, conv1_W, conv1_b, conv2_W,
           conv2_b, W1_w, W1_b, W2_w, W2_b):
    nt2 = node_types.astype(jnp.int32).reshape(NG * NN // 128, 128)
    edges = edge_indices.astype(jnp.int32)
    b1 = conv1_b.reshape(1, EMB_D)
    b2 = conv2_b.reshape(1, EMB_D)
    accs = []
    for half in range(NHALF):
        xraw = _GATHER_K[half](nt2, id_embed)
        cmat = _ADJ_K[half](edges).reshape(HGN, NN // 2, NN)
        accs.append(
            _tc_half(half, xraw, cmat, conv1_W, b1, conv2_W, b2, W1_w))
    out = _head(jnp.concatenate(accs, axis=0), W1_b.reshape(1, EMB_D), W2_w,
                W2_b.reshape(1, 1))
    return jnp.squeeze(out)
